# baseline (speedup 1.0000x reference)
import functools

import jax
import jax.numpy as jnp
from jax import lax
from jax.experimental import pallas as pl
from jax.experimental.pallas import tpu as pltpu
from jax.experimental.pallas import tpu_sc as plsc

_B, _N, _E, _D, _CH = 4, 10000, 320000, 128, 32
_NTILES = 16
_NCORES = 2
_CHUNK = 128
_NPAD = 10240
_PADROW = _N
_RPT = _NPAD // _NTILES
_CPY = 128
_EPT = _E // _NTILES
_FULL = _EPT // _CHUNK
_TAILPAD = _CHUNK - (_EPT - _FULL * _CHUNK)
_SLOTS = _FULL + 1


def _segsum_body(h_hbm, src_hbm, dst_hbm, zeros_hbm, out_hbm,
                 r0b, r1b, sidx4, didx4,
                 g0, g1, s0, s1, i0, i1, i2, i3, agg):
    c = lax.axis_index("c")
    t = lax.axis_index("s")
    rows = [r0b, r1b]
    gsem = [g0, g1]
    ssem = [s0, s1]
    isem = [i0, i1, i2, i3]
    ebase = t * _EPT
    pad16 = jnp.full((16,), _PADROW, jnp.int32)

    def ifire(slot, iu):
        base = pl.multiple_of(
            ebase + lax.min(slot * _CHUNK, _EPT - _CHUNK), 8)
        pltpu.async_copy(src_hbm.at[pl.ds(base, _CHUNK)], sidx4.at[iu],
                         isem[iu])
        pltpu.async_copy(dst_hbm.at[pl.ds(base, _CHUNK)], didx4.at[iu],
                         isem[iu])

    def iwait(iu):
        pltpu.make_async_copy(src_hbm.at[pl.ds(ebase, _CHUNK)],
                              sidx4.at[iu], isem[iu]).wait()
        pltpu.make_async_copy(dst_hbm.at[pl.ds(ebase, _CHUNK)],
                              didx4.at[iu], isem[iu]).wait()

    def ifix(slot, iu, off):
        for k in range(_CHUNK // 16):
            sidx4[iu, pl.ds(k * 16, 16)] = \
                sidx4[iu, pl.ds(k * 16, 16)] + off

        def padfix():
            for k in range(_TAILPAD // 16):
                didx4[iu, pl.ds(k * 16, 16)] = pad16

        if isinstance(slot, int):
            if slot == _FULL:
                padfix()
        else:
            pl.when(slot == _FULL)(padfix)

    def gather(u, iu):
        pltpu.async_copy(h_hbm.at[sidx4.at[iu]], rows[u], gsem[u])

    def gwait(u):
        pltpu.make_async_copy(h_hbm.at[sidx4.at[0]], rows[u],
                              gsem[u]).wait()

    def scatter(u, iu):
        pltpu.async_copy(rows[u], agg.at[didx4.at[iu]], ssem[u], add=True)

    def swait(u):
        pltpu.make_async_copy(rows[u], agg.at[didx4.at[0]],
                              ssem[u]).wait()

    if True:
        off = c * _N

        pltpu.async_copy(zeros_hbm, agg.at[pl.ds(t * _RPT, _RPT)], ssem[0])

        ifire(0, 0)
        ifire(1, 1)
        ifire(2, 2)
        iwait(0)
        ifix(0, 0, off)
        gather(0, 0)

        ifire(3, 3)
        iwait(1)
        ifix(1, 1, off)
        gather(1, 1)

        pltpu.make_async_copy(zeros_hbm, agg.at[pl.ds(t * _RPT, _RPT)],
                              ssem[0]).wait()
        plsc.subcore_barrier()
        gwait(0)
        scatter(0, 0)

        @pl.loop(0, (_SLOTS - 5) // 4)
        def _(jg):
            for v in range(4):
                j = 2 + jg * 4 + v
                u, up = v % 2, (v + 1) % 2
                iu, iup, iuf = (2 + v) % 4, (1 + v) % 4, v % 4
                swait(u)
                ifire(j + 2, iuf)
                iwait(iu)
                ifix(j, iu, off)
                gather(u, iu)
                gwait(up)
                scatter(up, iup)

        swait(0)
        ifire(_SLOTS - 1, 0)
        iwait(2)
        ifix(_SLOTS - 3, 2, off)
        gather(0, 2)
        gwait(1)
        scatter(1, 1)

        swait(1)
        iwait(3)
        ifix(_SLOTS - 2, 3, off)
        gather(1, 3)
        gwait(0)
        scatter(0, 2)

        swait(0)
        iwait(0)
        ifix(_SLOTS - 1, 0, off)
        gather(0, 0)
        gwait(1)
        scatter(1, 3)

        gwait(0)
        scatter(0, 0)
        swait(1)
        swait(0)
        plsc.subcore_barrier()

        @pl.when(t < _NTILES - 1)
        def _():
            pltpu.async_copy(agg.at[pl.ds(t * _RPT, _RPT)],
                             out_hbm.at[pl.ds(off + t * _RPT, _RPT)],
                             ssem[0])
            pltpu.make_async_copy(agg.at[pl.ds(t * _RPT, _RPT)],
                                  out_hbm.at[pl.ds(off, _RPT)],
                                  ssem[0]).wait()

        @pl.when(t == _NTILES - 1)
        def _():
            tail0 = (_NTILES - 1) * _RPT
            trows = _N - tail0
            pltpu.async_copy(agg.at[pl.ds(tail0, trows)],
                             out_hbm.at[pl.ds(off + tail0, trows)],
                             ssem[0])
            pltpu.make_async_copy(agg.at[pl.ds(tail0, trows)],
                                  out_hbm.at[pl.ds(off, trows)],
                                  ssem[0]).wait()


def _segsum(hflat, src, dst, zeros):
    kern = pl.kernel(
        _segsum_body,
        out_type=jax.ShapeDtypeStruct((_NCORES * _N, _D), jnp.float32),
        mesh=plsc.VectorSubcoreMesh(core_axis_name="c", subcore_axis_name="s"),
        scratch_types=[
            pltpu.VMEM((_CHUNK, _D), jnp.float32),
            pltpu.VMEM((_CHUNK, _D), jnp.float32),
            pltpu.VMEM((4, _CHUNK), jnp.int32),
            pltpu.VMEM((4, _CHUNK), jnp.int32),
            pltpu.SemaphoreType.DMA,
            pltpu.SemaphoreType.DMA,
            pltpu.SemaphoreType.DMA,
            pltpu.SemaphoreType.DMA,
            pltpu.SemaphoreType.DMA,
            pltpu.SemaphoreType.DMA,
            pltpu.SemaphoreType.DMA,
            pltpu.SemaphoreType.DMA,
            pltpu.VMEM_SHARED((_NPAD, _D), jnp.float32),
        ],
    )
    return kern(hflat, src, dst, zeros)


_BLK = 2000


def _mlp_body(h_ref, agg_ref, w1_ref, b1_ref, w2_ref, b2_ref, o_ref):
    z = jnp.dot(h_ref[0] + agg_ref[0], w1_ref[...],
                preferred_element_type=jnp.float32) + b1_ref[...]
    z = jnp.maximum(z, 0.0)
    o_ref[0] = jnp.dot(z, w2_ref[...],
                       preferred_element_type=jnp.float32) + b2_ref[...]


def _mlp(h, agg, w1, b1, w2, b2):
    nb = h.shape[0]
    return pl.pallas_call(
        _mlp_body,
        grid=(nb, _N // _BLK),
        in_specs=[
            pl.BlockSpec((1, _BLK, _D), lambda b, i: (b, i, 0)),
            pl.BlockSpec((1, _BLK, _D), lambda b, i: (b, i, 0)),
            pl.BlockSpec((_D, _D), lambda b, i: (0, 0)),
            pl.BlockSpec((1, _D), lambda b, i: (0, 0)),
            pl.BlockSpec((_D, _D), lambda b, i: (0, 0)),
            pl.BlockSpec((1, _D), lambda b, i: (0, 0)),
        ],
        out_specs=pl.BlockSpec((1, _BLK, _D), lambda b, i: (b, i, 0)),
        out_shape=jax.ShapeDtypeStruct((nb, _N, _D), jnp.float32),
    )(h, agg, w1, b1.reshape(1, _D), w2, b2.reshape(1, _D))


def _head_body(h_ref, cand_ref, act_ref, ms_ref, aw1_ref, ab1_ref,
               aw2_ref, ab2_ref, aw3_ref, ab3_ref, cw1_ref, cb1_ref,
               cw2_ref, cb2_ref, lp_ref, ent_ref, v_ref):
    b = pl.program_id(0)
    hb = h_ref[0]
    g = jnp.mean(hb, axis=0, keepdims=True)
    base = (jnp.dot(g, aw1_ref[_D:2 * _D], preferred_element_type=jnp.float32)
            + jnp.dot(ms_ref[...], aw1_ref[2 * _D:3 * _D],
                      preferred_element_type=jnp.float32)
            + ab1_ref[...])
    s1 = jnp.maximum(jnp.dot(hb, aw1_ref[0:_D],
                             preferred_element_type=jnp.float32) + base, 0.0)
    s2 = jnp.maximum(jnp.dot(s1, aw2_ref[...],
                             preferred_element_type=jnp.float32)
                     + ab2_ref[...], 0.0)
    s = (jnp.dot(s2, aw3_ref[...], preferred_element_type=jnp.float32)
         + ab3_ref[0, 0]) * 10.0

    m0 = jnp.max(s)
    e0 = jnp.exp(s - m0)
    probs0 = e0 / jnp.sum(e0)

    neg_inf = jnp.float32(-jnp.inf)
    logits = jnp.where(cand_ref[0] > 0, probs0, neg_inf)
    m1 = jnp.max(logits)
    e1 = jnp.exp(logits - m1)
    z1 = jnp.sum(e1)
    logp = logits - (m1 + jnp.log(z1))

    a = act_ref[b]
    row = lax.broadcasted_iota(jnp.int32, (_N, 1), 0)
    lp = jnp.sum(jnp.where(row == a, logp, 0.0))
    p = e1 / z1
    fmin = jnp.finfo(jnp.float32).min
    ent = -jnp.sum(p * jnp.maximum(logp, fmin))

    hv = jnp.maximum(jnp.dot(g, cw1_ref[...],
                             preferred_element_type=jnp.float32)
                     + cb1_ref[...], 0.0)
    v = jnp.sum(jnp.dot(hv, cw2_ref[...],
                        preferred_element_type=jnp.float32)) + cb2_ref[0, 0]

    lp_ref[b] = lp
    ent_ref[b] = ent
    v_ref[b] = v


def _head(h, candidates, action, machine_state,
          aw1, ab1, aw2, ab2, aw3, ab3, cw1, cb1, cw2, cb2):
    nb = h.shape[0]
    full = lambda b: (b, 0, 0)
    w0 = lambda b: (0, 0)
    return pl.pallas_call(
        _head_body,
        grid=(nb,),
        in_specs=[
            pl.BlockSpec((1, _N, _D), full),
            pl.BlockSpec((1, _N, 1), full),
            pl.BlockSpec(memory_space=pltpu.SMEM),
            pl.BlockSpec((1, _D), w0),
            pl.BlockSpec((3 * _D, _D), w0),
            pl.BlockSpec((1, _D), w0),
            pl.BlockSpec((_D, _D), w0),
            pl.BlockSpec((1, _D), w0),
            pl.BlockSpec((_D, 1), w0),
            pl.BlockSpec((1, 1), w0),
            pl.BlockSpec((_D, _CH), w0),
            pl.BlockSpec((1, _CH), w0),
            pl.BlockSpec((_CH, 1), w0),
            pl.BlockSpec((1, 1), w0),
        ],
        out_specs=[
            pl.BlockSpec(memory_space=pltpu.SMEM),
            pl.BlockSpec(memory_space=pltpu.SMEM),
            pl.BlockSpec(memory_space=pltpu.SMEM),
        ],
        out_shape=[
            jax.ShapeDtypeStruct((nb,), jnp.float32),
            jax.ShapeDtypeStruct((nb,), jnp.float32),
            jax.ShapeDtypeStruct((nb,), jnp.float32),
        ],
    )(h, candidates.reshape(nb, _N, 1), action,
      machine_state.reshape(1, _D), aw1, ab1.reshape(1, _D),
      aw2, ab2.reshape(1, _D), aw3, ab3.reshape(1, 1),
      cw1, cb1.reshape(1, _CH), cw2, cb2.reshape(1, 1))


def kernel(x, edge_index, candidates, action, machine_state,
           g0w1, g0b1, g0w2, g0b2, g1w1, g1b1, g1w2, g1b2,
           g2w1, g2b1, g2w2, g2b2, aw1, ab1, aw2, ab2, aw3, ab3,
           cw1, cb1, cw2, cb2):
    src, dst = edge_index[0], edge_index[1]
    zeros = jnp.zeros((_RPT, _D), jnp.float32)

    pairs = [x[0:2], x[2:4]]
    for (w1, b1, w2, b2) in ((g0w1, g0b1, g0w2, g0b2),
                             (g1w1, g1b1, g1w2, g1b2),
                             (g2w1, g2b1, g2w2, g2b2)):
        aggs = [_segsum(h.reshape(2 * _N, _D), src, dst,
                        zeros).reshape(2, _N, _D)
                for h in pairs]
        pairs = [_mlp(h, agg, w1, b1, w2, b2)
                 for h, agg in zip(pairs, aggs)]

    outs = [_head(pairs[p], candidates[2 * p:2 * p + 2],
                  action[2 * p:2 * p + 2], machine_state,
                  aw1, ab1, aw2, ab2, aw3, ab3, cw1, cb1, cw2, cb2)
            for p in range(2)]
    lps = jnp.concatenate([outs[0][0], outs[1][0]])
    ents = jnp.concatenate([outs[0][1], outs[1][1]])
    vs = jnp.concatenate([outs[0][2], outs[1][2]])
    return action, lps, ents, vs


# speedup vs baseline: 1.0017x; 1.0017x over previous
import functools

import jax
import jax.numpy as jnp
from jax import lax
from jax.experimental import pallas as pl
from jax.experimental.pallas import tpu as pltpu
from jax.experimental.pallas import tpu_sc as plsc

_B, _N, _E, _D, _CH = 4, 10000, 320000, 128, 32
_NTILES = 16
_NCORES = 2
_CHUNK = 128
_NPAD = 10240
_PADROW = _N
_RPT = _NPAD // _NTILES
_CPY = 128
_EPT = _E // _NTILES
_FULL = _EPT // _CHUNK
_TAILPAD = _CHUNK - (_EPT - _FULL * _CHUNK)
_SLOTS = _FULL + 1


def _segsum_body(h_hbm, src_hbm, dst_hbm, zeros_hbm, out_hbm,
                 r0b, r1b, sidx4, didx4,
                 g0, g1, s0, s1, i0, i1, i2, i3, agg):
    c = lax.axis_index("c")
    t = lax.axis_index("s")
    rows = [r0b, r1b]
    gsem = [g0, g1]
    ssem = [s0, s1]
    isem = [i0, i1, i2, i3]
    ebase = t * _EPT
    pad16 = jnp.full((16,), _PADROW, jnp.int32)

    def ifire(slot, iu):
        base = pl.multiple_of(
            ebase + lax.min(slot * _CHUNK, _EPT - _CHUNK), 8)
        pltpu.async_copy(src_hbm.at[pl.ds(base, _CHUNK)], sidx4.at[iu],
                         isem[iu])
        pltpu.async_copy(dst_hbm.at[pl.ds(base, _CHUNK)], didx4.at[iu],
                         isem[iu])

    def iwait(iu):
        pltpu.make_async_copy(src_hbm.at[pl.ds(ebase, _CHUNK)],
                              sidx4.at[iu], isem[iu]).wait()
        pltpu.make_async_copy(dst_hbm.at[pl.ds(ebase, _CHUNK)],
                              didx4.at[iu], isem[iu]).wait()

    def ifix(slot, iu, off):
        for k in range(_CHUNK // 16):
            sidx4[iu, pl.ds(k * 16, 16)] = \
                sidx4[iu, pl.ds(k * 16, 16)] + off

        def padfix():
            for k in range(_TAILPAD // 16):
                didx4[iu, pl.ds(k * 16, 16)] = pad16

        if isinstance(slot, int):
            if slot == _FULL:
                padfix()
        else:
            pl.when(slot == _FULL)(padfix)

    def gather(u, iu):
        pltpu.async_copy(h_hbm.at[sidx4.at[iu]], rows[u], gsem[u])

    def gwait(u):
        pltpu.make_async_copy(h_hbm.at[sidx4.at[0]], rows[u],
                              gsem[u]).wait()

    def scatter(u, iu):
        pltpu.async_copy(rows[u], agg.at[didx4.at[iu]], ssem[u], add=True)

    def swait(u):
        pltpu.make_async_copy(rows[u], agg.at[didx4.at[0]],
                              ssem[u]).wait()

    if True:
        off = c * _N

        for i in range(_RPT // _CPY):
            pltpu.async_copy(zeros_hbm.at[pl.ds(i * _CPY, _CPY)],
                             agg.at[pl.ds(t * _RPT + i * _CPY, _CPY)],
                             ssem[0])

        ifire(0, 0)
        ifire(1, 1)
        ifire(2, 2)
        iwait(0)
        ifix(0, 0, off)
        gather(0, 0)

        ifire(3, 3)
        iwait(1)
        ifix(1, 1, off)
        gather(1, 1)

        for i in range(_RPT // _CPY):
            pltpu.make_async_copy(zeros_hbm.at[pl.ds(0, _CPY)],
                                  agg.at[pl.ds(t * _RPT, _CPY)],
                                  ssem[0]).wait()
        plsc.subcore_barrier()
        gwait(0)
        scatter(0, 0)

        @pl.loop(0, (_SLOTS - 5) // 4)
        def _(jg):
            for v in range(4):
                j = 2 + jg * 4 + v
                u, up = v % 2, (v + 1) % 2
                iu, iup, iuf = (2 + v) % 4, (1 + v) % 4, v % 4
                swait(u)
                ifire(j + 2, iuf)
                iwait(iu)
                ifix(j, iu, off)
                gather(u, iu)
                gwait(up)
                scatter(up, iup)

        swait(0)
        ifire(_SLOTS - 1, 0)
        iwait(2)
        ifix(_SLOTS - 3, 2, off)
        gather(0, 2)
        gwait(1)
        scatter(1, 1)

        swait(1)
        iwait(3)
        ifix(_SLOTS - 2, 3, off)
        gather(1, 3)
        gwait(0)
        scatter(0, 2)

        swait(0)
        iwait(0)
        ifix(_SLOTS - 1, 0, off)
        gather(0, 0)
        gwait(1)
        scatter(1, 3)

        gwait(0)
        scatter(0, 0)
        swait(1)
        swait(0)
        plsc.subcore_barrier()

        @pl.when(t < _NTILES - 1)
        def _():
            for i in range(_RPT // _CPY):
                u = i % 2
                if i >= 2:
                    pltpu.make_async_copy(
                        rows[u], out_hbm.at[pl.ds(off, _CPY)],
                        ssem[u]).wait()
                rr = t * _RPT + i * _CPY
                pltpu.async_copy(agg.at[pl.ds(rr, _CPY)], rows[u], gsem[u])
                pltpu.make_async_copy(agg.at[pl.ds(rr, _CPY)], rows[u],
                                      gsem[u]).wait()
                pltpu.async_copy(rows[u], out_hbm.at[pl.ds(off + rr, _CPY)],
                                 ssem[u])
            pltpu.make_async_copy(rows[1], out_hbm.at[pl.ds(off, _CPY)],
                                  ssem[1]).wait()
            pltpu.make_async_copy(rows[0], out_hbm.at[pl.ds(off, _CPY)],
                                  ssem[0]).wait()

        @pl.when(t == _NTILES - 1)
        def _():
            tail0 = (_NTILES - 1) * _RPT
            tcpy = (_N - tail0) // 5
            for i in range(5):
                rr = tail0 + i * tcpy
                pltpu.sync_copy(agg.at[pl.ds(rr, tcpy)],
                                rows[0].at[pl.ds(0, tcpy)])
                pltpu.sync_copy(rows[0].at[pl.ds(0, tcpy)],
                                out_hbm.at[pl.ds(off + rr, tcpy)])


def _segsum(hflat, src, dst, zeros):
    kern = pl.kernel(
        _segsum_body,
        out_type=jax.ShapeDtypeStruct((_NCORES * _N, _D), jnp.float32),
        mesh=plsc.VectorSubcoreMesh(core_axis_name="c", subcore_axis_name="s"),
        scratch_types=[
            pltpu.VMEM((_CHUNK, _D), jnp.float32),
            pltpu.VMEM((_CHUNK, _D), jnp.float32),
            pltpu.VMEM((4, _CHUNK), jnp.int32),
            pltpu.VMEM((4, _CHUNK), jnp.int32),
            pltpu.SemaphoreType.DMA,
            pltpu.SemaphoreType.DMA,
            pltpu.SemaphoreType.DMA,
            pltpu.SemaphoreType.DMA,
            pltpu.SemaphoreType.DMA,
            pltpu.SemaphoreType.DMA,
            pltpu.SemaphoreType.DMA,
            pltpu.SemaphoreType.DMA,
            pltpu.VMEM_SHARED((_NPAD, _D), jnp.float32),
        ],
    )
    return kern(hflat, src, dst, zeros)


_BLK = 2000


def _mlp_body(h_ref, agg_ref, w1_ref, b1_ref, w2_ref, b2_ref, o_ref):
    z = jnp.dot(h_ref[0] + agg_ref[0], w1_ref[...],
                preferred_element_type=jnp.float32) + b1_ref[...]
    z = jnp.maximum(z, 0.0)
    o_ref[0] = jnp.dot(z, w2_ref[...],
                       preferred_element_type=jnp.float32) + b2_ref[...]


def _mlp(h, agg, w1, b1, w2, b2):
    nb = h.shape[0]
    return pl.pallas_call(
        _mlp_body,
        grid=(nb, _N // _BLK),
        in_specs=[
            pl.BlockSpec((1, _BLK, _D), lambda b, i: (b, i, 0)),
            pl.BlockSpec((1, _BLK, _D), lambda b, i: (b, i, 0)),
            pl.BlockSpec((_D, _D), lambda b, i: (0, 0)),
            pl.BlockSpec((1, _D), lambda b, i: (0, 0)),
            pl.BlockSpec((_D, _D), lambda b, i: (0, 0)),
            pl.BlockSpec((1, _D), lambda b, i: (0, 0)),
        ],
        out_specs=pl.BlockSpec((1, _BLK, _D), lambda b, i: (b, i, 0)),
        out_shape=jax.ShapeDtypeStruct((nb, _N, _D), jnp.float32),
    )(h, agg, w1, b1.reshape(1, _D), w2, b2.reshape(1, _D))


def _head_body(h_ref, cand_ref, act_ref, ms_ref, aw1_ref, ab1_ref,
               aw2_ref, ab2_ref, aw3_ref, ab3_ref, cw1_ref, cb1_ref,
               cw2_ref, cb2_ref, lp_ref, ent_ref, v_ref):
    b = pl.program_id(0)
    hb = h_ref[0]
    g = jnp.mean(hb, axis=0, keepdims=True)
    base = (jnp.dot(g, aw1_ref[_D:2 * _D], preferred_element_type=jnp.float32)
            + jnp.dot(ms_ref[...], aw1_ref[2 * _D:3 * _D],
                      preferred_element_type=jnp.float32)
            + ab1_ref[...])
    s1 = jnp.maximum(jnp.dot(hb, aw1_ref[0:_D],
                             preferred_element_type=jnp.float32) + base, 0.0)
    s2 = jnp.maximum(jnp.dot(s1, aw2_ref[...],
                             preferred_element_type=jnp.float32)
                     + ab2_ref[...], 0.0)
    s = (jnp.dot(s2, aw3_ref[...], preferred_element_type=jnp.float32)
         + ab3_ref[0, 0]) * 10.0

    m0 = jnp.max(s)
    e0 = jnp.exp(s - m0)
    probs0 = e0 / jnp.sum(e0)

    neg_inf = jnp.float32(-jnp.inf)
    logits = jnp.where(cand_ref[0] > 0, probs0, neg_inf)
    m1 = jnp.max(logits)
    e1 = jnp.exp(logits - m1)
    z1 = jnp.sum(e1)
    logp = logits - (m1 + jnp.log(z1))

    a = act_ref[b]
    row = lax.broadcasted_iota(jnp.int32, (_N, 1), 0)
    lp = jnp.sum(jnp.where(row == a, logp, 0.0))
    p = e1 / z1
    fmin = jnp.finfo(jnp.float32).min
    ent = -jnp.sum(p * jnp.maximum(logp, fmin))

    hv = jnp.maximum(jnp.dot(g, cw1_ref[...],
                             preferred_element_type=jnp.float32)
                     + cb1_ref[...], 0.0)
    v = jnp.sum(jnp.dot(hv, cw2_ref[...],
                        preferred_element_type=jnp.float32)) + cb2_ref[0, 0]

    lp_ref[b] = lp
    ent_ref[b] = ent
    v_ref[b] = v


def _head(h, candidates, action, machine_state,
          aw1, ab1, aw2, ab2, aw3, ab3, cw1, cb1, cw2, cb2):
    nb = h.shape[0]
    full = lambda b: (b, 0, 0)
    w0 = lambda b: (0, 0)
    return pl.pallas_call(
        _head_body,
        grid=(nb,),
        in_specs=[
            pl.BlockSpec((1, _N, _D), full),
            pl.BlockSpec((1, _N, 1), full),
            pl.BlockSpec(memory_space=pltpu.SMEM),
            pl.BlockSpec((1, _D), w0),
            pl.BlockSpec((3 * _D, _D), w0),
            pl.BlockSpec((1, _D), w0),
            pl.BlockSpec((_D, _D), w0),
            pl.BlockSpec((1, _D), w0),
            pl.BlockSpec((_D, 1), w0),
            pl.BlockSpec((1, 1), w0),
            pl.BlockSpec((_D, _CH), w0),
            pl.BlockSpec((1, _CH), w0),
            pl.BlockSpec((_CH, 1), w0),
            pl.BlockSpec((1, 1), w0),
        ],
        out_specs=[
            pl.BlockSpec(memory_space=pltpu.SMEM),
            pl.BlockSpec(memory_space=pltpu.SMEM),
            pl.BlockSpec(memory_space=pltpu.SMEM),
        ],
        out_shape=[
            jax.ShapeDtypeStruct((nb,), jnp.float32),
            jax.ShapeDtypeStruct((nb,), jnp.float32),
            jax.ShapeDtypeStruct((nb,), jnp.float32),
        ],
    )(h, candidates.reshape(nb, _N, 1), action,
      machine_state.reshape(1, _D), aw1, ab1.reshape(1, _D),
      aw2, ab2.reshape(1, _D), aw3, ab3.reshape(1, 1),
      cw1, cb1.reshape(1, _CH), cw2, cb2.reshape(1, 1))


def kernel(x, edge_index, candidates, action, machine_state,
           g0w1, g0b1, g0w2, g0b2, g1w1, g1b1, g1w2, g1b2,
           g2w1, g2b1, g2w2, g2b2, aw1, ab1, aw2, ab2, aw3, ab3,
           cw1, cb1, cw2, cb2):
    src, dst = edge_index[0], edge_index[1]
    zeros = jnp.zeros((_RPT, _D), jnp.float32)

    pairs = [x[0:2], x[2:4]]
    for (w1, b1, w2, b2) in ((g0w1, g0b1, g0w2, g0b2),
                             (g1w1, g1b1, g1w2, g1b2)):
        aggs = [_segsum(h.reshape(2 * _N, _D), src, dst,
                        zeros).reshape(2, _N, _D)
                for h in pairs]
        pairs = [_mlp(h, agg, w1, b1, w2, b2)
                 for h, agg in zip(pairs, aggs)]

    aggs = [_segsum(h.reshape(2 * _N, _D), src, dst,
                    zeros).reshape(2, _N, _D)
            for h in pairs]
    outs = []
    for p in range(2):
        hp = _mlp(pairs[p], aggs[p], g2w1, g2b1, g2w2, g2b2)
        outs.append(_head(hp, candidates[2 * p:2 * p + 2],
                          action[2 * p:2 * p + 2], machine_state,
                          aw1, ab1, aw2, ab2, aw3, ab3, cw1, cb1, cw2, cb2))
    lps = jnp.concatenate([outs[0][0], outs[1][0]])
    ents = jnp.concatenate([outs[0][1], outs[1][1]])
    vs = jnp.concatenate([outs[0][2], outs[1][2]])
    return action, lps, ents, vs


# speedup vs baseline: 1.0265x; 1.0248x over previous
import functools

import jax
import jax.numpy as jnp
from jax import lax
from jax.experimental import pallas as pl
from jax.experimental.pallas import tpu as pltpu
from jax.experimental.pallas import tpu_sc as plsc

_B, _N, _E, _D, _CH = 4, 10000, 320000, 128, 32
_NTILES = 16
_NCORES = 2
_CHUNK = 128
_NPAD = 10240
_PADROW = _N
_RPT = _NPAD // _NTILES
_CPY = 128
_EPT = _E // _NTILES
_FULL = _EPT // _CHUNK
_TAILPAD = _CHUNK - (_EPT - _FULL * _CHUNK)
_SLOTS = _FULL + 1


def _segsum_body(h_hbm, src_hbm, dst_hbm, out_hbm,
                 r0b, r1b, sidx4, didx4,
                 g0, g1, s0, s1, i0, i1, i2, i3, agg):
    c = lax.axis_index("c")
    t = lax.axis_index("s")
    rows = [r0b, r1b]
    gsem = [g0, g1]
    ssem = [s0, s1]
    isem = [i0, i1, i2, i3]
    ebase = t * _EPT
    pad16 = jnp.full((16,), _PADROW, jnp.int32)
    zero16 = jnp.zeros((16,), jnp.float32)

    def ifire(slot, iu):
        base = pl.multiple_of(
            ebase + lax.min(slot * _CHUNK, _EPT - _CHUNK), 8)
        pltpu.async_copy(src_hbm.at[pl.ds(base, _CHUNK)], sidx4.at[iu],
                         isem[iu])
        pltpu.async_copy(dst_hbm.at[pl.ds(base, _CHUNK)], didx4.at[iu],
                         isem[iu])

    def iwait(iu):
        pltpu.make_async_copy(src_hbm.at[pl.ds(ebase, _CHUNK)],
                              sidx4.at[iu], isem[iu]).wait()
        pltpu.make_async_copy(dst_hbm.at[pl.ds(ebase, _CHUNK)],
                              didx4.at[iu], isem[iu]).wait()

    def ifix(slot, iu, off):
        for k in range(_CHUNK // 16):
            sidx4[iu, pl.ds(k * 16, 16)] = \
                sidx4[iu, pl.ds(k * 16, 16)] + off

        def padfix():
            for k in range(_TAILPAD // 16):
                didx4[iu, pl.ds(k * 16, 16)] = pad16

        if isinstance(slot, int):
            if slot == _FULL:
                padfix()
        else:
            pl.when(slot == _FULL)(padfix)

    def gather(u, iu):
        pltpu.async_copy(h_hbm.at[sidx4.at[iu]], rows[u], gsem[u])

    def gwait(u):
        pltpu.make_async_copy(h_hbm.at[sidx4.at[0]], rows[u],
                              gsem[u]).wait()

    def scatter(u, iu):
        pltpu.async_copy(rows[u], agg.at[didx4.at[iu]], ssem[u], add=True)

    def swait(u):
        pltpu.make_async_copy(rows[u], agg.at[didx4.at[0]],
                              ssem[u]).wait()

    if True:
        off = c * _N

        @pl.loop(0, _CPY)
        def _(r):
            for k in range(_D // 16):
                rows[0][r, pl.ds(k * 16, 16)] = zero16

        for i in range(_RPT // _CPY):
            pltpu.async_copy(rows[0],
                             agg.at[pl.ds(t * _RPT + i * _CPY, _CPY)],
                             gsem[0])
        for i in range(_RPT // _CPY):
            pltpu.make_async_copy(rows[0], agg.at[pl.ds(t * _RPT, _CPY)],
                                  gsem[0]).wait()
        plsc.subcore_barrier()

        ifire(0, 0)
        ifire(1, 1)
        ifire(2, 2)
        iwait(0)
        ifix(0, 0, off)
        gather(0, 0)

        ifire(3, 3)
        iwait(1)
        ifix(1, 1, off)
        gather(1, 1)
        gwait(0)
        scatter(0, 0)

        @pl.loop(0, (_SLOTS - 5) // 4)
        def _(jg):
            for v in range(4):
                j = 2 + jg * 4 + v
                u, up = v % 2, (v + 1) % 2
                iu, iup, iuf = (2 + v) % 4, (1 + v) % 4, v % 4
                swait(u)
                ifire(j + 2, iuf)
                iwait(iu)
                ifix(j, iu, off)
                gather(u, iu)
                gwait(up)
                scatter(up, iup)

        swait(0)
        ifire(_SLOTS - 1, 0)
        iwait(2)
        ifix(_SLOTS - 3, 2, off)
        gather(0, 2)
        gwait(1)
        scatter(1, 1)

        swait(1)
        iwait(3)
        ifix(_SLOTS - 2, 3, off)
        gather(1, 3)
        gwait(0)
        scatter(0, 2)

        swait(0)
        iwait(0)
        ifix(_SLOTS - 1, 0, off)
        gather(0, 0)
        gwait(1)
        scatter(1, 3)

        gwait(0)
        scatter(0, 0)
        swait(1)
        swait(0)
        plsc.subcore_barrier()

        @pl.when(t < _NTILES - 1)
        def _():
            for i in range(_RPT // _CPY):
                u = i % 2
                if i >= 2:
                    pltpu.make_async_copy(
                        rows[u], out_hbm.at[pl.ds(off, _CPY)],
                        ssem[u]).wait()
                rr = t * _RPT + i * _CPY
                pltpu.async_copy(agg.at[pl.ds(rr, _CPY)], rows[u], gsem[u])
                pltpu.make_async_copy(agg.at[pl.ds(rr, _CPY)], rows[u],
                                      gsem[u]).wait()
                pltpu.async_copy(rows[u], out_hbm.at[pl.ds(off + rr, _CPY)],
                                 ssem[u])
            pltpu.make_async_copy(rows[1], out_hbm.at[pl.ds(off, _CPY)],
                                  ssem[1]).wait()
            pltpu.make_async_copy(rows[0], out_hbm.at[pl.ds(off, _CPY)],
                                  ssem[0]).wait()

        @pl.when(t == _NTILES - 1)
        def _():
            tail0 = (_NTILES - 1) * _RPT
            tcpy = (_N - tail0) // 5
            for i in range(5):
                rr = tail0 + i * tcpy
                pltpu.sync_copy(agg.at[pl.ds(rr, tcpy)],
                                rows[0].at[pl.ds(0, tcpy)])
                pltpu.sync_copy(rows[0].at[pl.ds(0, tcpy)],
                                out_hbm.at[pl.ds(off + rr, tcpy)])


def _segsum(hflat, src, dst):
    kern = pl.kernel(
        _segsum_body,
        out_type=jax.ShapeDtypeStruct((_NCORES * _N, _D), jnp.float32),
        mesh=plsc.VectorSubcoreMesh(core_axis_name="c", subcore_axis_name="s"),
        scratch_types=[
            pltpu.VMEM((_CHUNK, _D), jnp.float32),
            pltpu.VMEM((_CHUNK, _D), jnp.float32),
            pltpu.VMEM((4, _CHUNK), jnp.int32),
            pltpu.VMEM((4, _CHUNK), jnp.int32),
            pltpu.SemaphoreType.DMA,
            pltpu.SemaphoreType.DMA,
            pltpu.SemaphoreType.DMA,
            pltpu.SemaphoreType.DMA,
            pltpu.SemaphoreType.DMA,
            pltpu.SemaphoreType.DMA,
            pltpu.SemaphoreType.DMA,
            pltpu.SemaphoreType.DMA,
            pltpu.VMEM_SHARED((_NPAD, _D), jnp.float32),
        ],
    )
    return kern(hflat, src, dst)


_BLK = 2000


def _mlp_body(h_ref, agg_ref, w1_ref, b1_ref, w2_ref, b2_ref, o_ref):
    z = jnp.dot(h_ref[0] + agg_ref[0], w1_ref[...],
                preferred_element_type=jnp.float32) + b1_ref[...]
    z = jnp.maximum(z, 0.0)
    o_ref[0] = jnp.dot(z, w2_ref[...],
                       preferred_element_type=jnp.float32) + b2_ref[...]


def _mlp(h, agg, w1, b1, w2, b2):
    nb = h.shape[0]
    return pl.pallas_call(
        _mlp_body,
        grid=(nb, _N // _BLK),
        in_specs=[
            pl.BlockSpec((1, _BLK, _D), lambda b, i: (b, i, 0)),
            pl.BlockSpec((1, _BLK, _D), lambda b, i: (b, i, 0)),
            pl.BlockSpec((_D, _D), lambda b, i: (0, 0)),
            pl.BlockSpec((1, _D), lambda b, i: (0, 0)),
            pl.BlockSpec((_D, _D), lambda b, i: (0, 0)),
            pl.BlockSpec((1, _D), lambda b, i: (0, 0)),
        ],
        out_specs=pl.BlockSpec((1, _BLK, _D), lambda b, i: (b, i, 0)),
        out_shape=jax.ShapeDtypeStruct((nb, _N, _D), jnp.float32),
    )(h, agg, w1, b1.reshape(1, _D), w2, b2.reshape(1, _D))


def _head_body(h_ref, cand_ref, act_ref, ms_ref, aw1_ref, ab1_ref,
               aw2_ref, ab2_ref, aw3_ref, ab3_ref, cw1_ref, cb1_ref,
               cw2_ref, cb2_ref, lp_ref, ent_ref, v_ref):
    b = pl.program_id(0)
    hb = h_ref[0]
    g = jnp.mean(hb, axis=0, keepdims=True)
    base = (jnp.dot(g, aw1_ref[_D:2 * _D], preferred_element_type=jnp.float32)
            + jnp.dot(ms_ref[...], aw1_ref[2 * _D:3 * _D],
                      preferred_element_type=jnp.float32)
            + ab1_ref[...])
    s1 = jnp.maximum(jnp.dot(hb, aw1_ref[0:_D],
                             preferred_element_type=jnp.float32) + base, 0.0)
    s2 = jnp.maximum(jnp.dot(s1, aw2_ref[...],
                             preferred_element_type=jnp.float32)
                     + ab2_ref[...], 0.0)
    s = (jnp.dot(s2, aw3_ref[...], preferred_element_type=jnp.float32)
         + ab3_ref[0, 0]) * 10.0

    m0 = jnp.max(s)
    e0 = jnp.exp(s - m0)
    probs0 = e0 / jnp.sum(e0)

    neg_inf = jnp.float32(-jnp.inf)
    logits = jnp.where(cand_ref[0] > 0, probs0, neg_inf)
    m1 = jnp.max(logits)
    e1 = jnp.exp(logits - m1)
    z1 = jnp.sum(e1)
    logp = logits - (m1 + jnp.log(z1))

    a = act_ref[b]
    row = lax.broadcasted_iota(jnp.int32, (_N, 1), 0)
    lp = jnp.sum(jnp.where(row == a, logp, 0.0))
    p = e1 / z1
    fmin = jnp.finfo(jnp.float32).min
    ent = -jnp.sum(p * jnp.maximum(logp, fmin))

    hv = jnp.maximum(jnp.dot(g, cw1_ref[...],
                             preferred_element_type=jnp.float32)
                     + cb1_ref[...], 0.0)
    v = jnp.sum(jnp.dot(hv, cw2_ref[...],
                        preferred_element_type=jnp.float32)) + cb2_ref[0, 0]

    lp_ref[b] = lp
    ent_ref[b] = ent
    v_ref[b] = v


def _head(h, candidates, action, machine_state,
          aw1, ab1, aw2, ab2, aw3, ab3, cw1, cb1, cw2, cb2):
    nb = h.shape[0]
    full = lambda b: (b, 0, 0)
    w0 = lambda b: (0, 0)
    return pl.pallas_call(
        _head_body,
        grid=(nb,),
        in_specs=[
            pl.BlockSpec((1, _N, _D), full),
            pl.BlockSpec((1, _N, 1), full),
            pl.BlockSpec(memory_space=pltpu.SMEM),
            pl.BlockSpec((1, _D), w0),
            pl.BlockSpec((3 * _D, _D), w0),
            pl.BlockSpec((1, _D), w0),
            pl.BlockSpec((_D, _D), w0),
            pl.BlockSpec((1, _D), w0),
            pl.BlockSpec((_D, 1), w0),
            pl.BlockSpec((1, 1), w0),
            pl.BlockSpec((_D, _CH), w0),
            pl.BlockSpec((1, _CH), w0),
            pl.BlockSpec((_CH, 1), w0),
            pl.BlockSpec((1, 1), w0),
        ],
        out_specs=[
            pl.BlockSpec(memory_space=pltpu.SMEM),
            pl.BlockSpec(memory_space=pltpu.SMEM),
            pl.BlockSpec(memory_space=pltpu.SMEM),
        ],
        out_shape=[
            jax.ShapeDtypeStruct((nb,), jnp.float32),
            jax.ShapeDtypeStruct((nb,), jnp.float32),
            jax.ShapeDtypeStruct((nb,), jnp.float32),
        ],
    )(h, candidates.reshape(nb, _N, 1), action,
      machine_state.reshape(1, _D), aw1, ab1.reshape(1, _D),
      aw2, ab2.reshape(1, _D), aw3, ab3.reshape(1, 1),
      cw1, cb1.reshape(1, _CH), cw2, cb2.reshape(1, 1))


def kernel(x, edge_index, candidates, action, machine_state,
           g0w1, g0b1, g0w2, g0b2, g1w1, g1b1, g1w2, g1b2,
           g2w1, g2b1, g2w2, g2b2, aw1, ab1, aw2, ab2, aw3, ab3,
           cw1, cb1, cw2, cb2):
    src, dst = edge_index[0], edge_index[1]

    pairs = [x[0:2], x[2:4]]
    for (w1, b1, w2, b2) in ((g0w1, g0b1, g0w2, g0b2),
                             (g1w1, g1b1, g1w2, g1b2)):
        aggs = [_segsum(h.reshape(2 * _N, _D), src,
                        dst).reshape(2, _N, _D)
                for h in pairs]
        pairs = [_mlp(h, agg, w1, b1, w2, b2)
                 for h, agg in zip(pairs, aggs)]

    aggs = [_segsum(h.reshape(2 * _N, _D), src,
                    dst).reshape(2, _N, _D)
            for h in pairs]
    outs = []
    for p in range(2):
        hp = _mlp(pairs[p], aggs[p], g2w1, g2b1, g2w2, g2b2)
        outs.append(_head(hp, candidates[2 * p:2 * p + 2],
                          action[2 * p:2 * p + 2], machine_state,
                          aw1, ab1, aw2, ab2, aw3, ab3, cw1, cb1, cw2, cb2))
    lps = jnp.concatenate([outs[0][0], outs[1][0]])
    ents = jnp.concatenate([outs[0][1], outs[1][1]])
    vs = jnp.concatenate([outs[0][2], outs[1][2]])
    return action, lps, ents, vs


# speedup vs baseline: 1.0621x; 1.0346x over previous
import functools

import jax
import jax.numpy as jnp
from jax import lax
from jax.experimental import pallas as pl
from jax.experimental.pallas import tpu as pltpu
from jax.experimental.pallas import tpu_sc as plsc

_B, _N, _E, _D, _CH = 4, 10000, 320000, 128, 32
_NTILES = 16
_NCORES = 2
_CHUNK = 128
_NPAD = 10240
_PADROW = _N
_RPT = _NPAD // _NTILES
_CPY = 128
_EPT = _E // _NTILES
_FULL = _EPT // _CHUNK
_TAILPAD = _CHUNK - (_EPT - _FULL * _CHUNK)
_SLOTS = _FULL + 1


def _segsum_body(h_hbm, src_hbm, dst_hbm, out_hbm,
                 r0b, r1b, sidx4, didx4,
                 g0, g1, s0, s1, i0, i1, i2, i3, agg):
    c = lax.axis_index("c")
    t = lax.axis_index("s")
    rows = [r0b, r1b]
    gsem = [g0, g1]
    ssem = [s0, s1]
    isem = [i0, i1, i2, i3]
    ebase = t * _EPT
    pad16 = jnp.full((16,), _PADROW, jnp.int32)
    zero16 = jnp.zeros((16,), jnp.float32)

    def ifire(slot, iu):
        base = pl.multiple_of(
            ebase + lax.min(slot * _CHUNK, _EPT - _CHUNK), 8)
        pltpu.async_copy(src_hbm.at[pl.ds(base, _CHUNK)], sidx4.at[iu],
                         isem[iu])
        pltpu.async_copy(dst_hbm.at[pl.ds(base, _CHUNK)], didx4.at[iu],
                         isem[iu])

    def iwait(iu):
        pltpu.make_async_copy(src_hbm.at[pl.ds(ebase, _CHUNK)],
                              sidx4.at[iu], isem[iu]).wait()
        pltpu.make_async_copy(dst_hbm.at[pl.ds(ebase, _CHUNK)],
                              didx4.at[iu], isem[iu]).wait()

    def ifix(slot, iu, off):
        for k in range(_CHUNK // 16):
            sidx4[iu, pl.ds(k * 16, 16)] = \
                sidx4[iu, pl.ds(k * 16, 16)] + off

        def padfix():
            for k in range(_TAILPAD // 16):
                didx4[iu, pl.ds(k * 16, 16)] = pad16

        if isinstance(slot, int):
            if slot == _FULL:
                padfix()
        else:
            pl.when(slot == _FULL)(padfix)

    def gather(u, iu):
        pltpu.async_copy(h_hbm.at[sidx4.at[iu]], rows[u], gsem[u])

    def gwait(u):
        pltpu.make_async_copy(h_hbm.at[sidx4.at[0]], rows[u],
                              gsem[u]).wait()

    def scatter(u, iu):
        pltpu.async_copy(rows[u], agg.at[didx4.at[iu]], ssem[u], add=True)

    def swait(u):
        pltpu.make_async_copy(rows[u], agg.at[didx4.at[0]],
                              ssem[u]).wait()

    if True:
        off = c * _N

        @pl.loop(0, _CPY)
        def _(r):
            for k in range(_D // 16):
                rows[0][r, pl.ds(k * 16, 16)] = zero16

        for i in range(_RPT // _CPY):
            pltpu.async_copy(rows[0],
                             agg.at[pl.ds(t * _RPT + i * _CPY, _CPY)],
                             gsem[0])
        for i in range(_RPT // _CPY):
            pltpu.make_async_copy(rows[0], agg.at[pl.ds(t * _RPT, _CPY)],
                                  gsem[0]).wait()
        plsc.subcore_barrier()

        ifire(0, 0)
        ifire(1, 1)
        ifire(2, 2)
        iwait(0)
        ifix(0, 0, off)
        gather(0, 0)

        ifire(3, 3)
        iwait(1)
        ifix(1, 1, off)
        gather(1, 1)
        gwait(0)
        scatter(0, 0)

        @pl.loop(0, (_SLOTS - 5) // 4)
        def _(jg):
            for v in range(4):
                j = 2 + jg * 4 + v
                u, up = v % 2, (v + 1) % 2
                iu, iup, iuf = (2 + v) % 4, (1 + v) % 4, v % 4
                swait(u)
                ifire(j + 2, iuf)
                iwait(iu)
                ifix(j, iu, off)
                gather(u, iu)
                gwait(up)
                scatter(up, iup)

        swait(0)
        ifire(_SLOTS - 1, 0)
        iwait(2)
        ifix(_SLOTS - 3, 2, off)
        gather(0, 2)
        gwait(1)
        scatter(1, 1)

        swait(1)
        iwait(3)
        ifix(_SLOTS - 2, 3, off)
        gather(1, 3)
        gwait(0)
        scatter(0, 2)

        swait(0)
        iwait(0)
        ifix(_SLOTS - 1, 0, off)
        gather(0, 0)
        gwait(1)
        scatter(1, 3)

        gwait(0)
        scatter(0, 0)
        swait(1)
        swait(0)
        plsc.subcore_barrier()

        @pl.when(t < _NTILES - 1)
        def _():
            for i in range(_RPT // _CPY):
                u = i % 2
                if i >= 2:
                    pltpu.make_async_copy(
                        rows[u], out_hbm.at[pl.ds(off, _CPY)],
                        ssem[u]).wait()
                rr = t * _RPT + i * _CPY
                pltpu.async_copy(agg.at[pl.ds(rr, _CPY)], rows[u], gsem[u])
                pltpu.make_async_copy(agg.at[pl.ds(rr, _CPY)], rows[u],
                                      gsem[u]).wait()
                pltpu.async_copy(rows[u], out_hbm.at[pl.ds(off + rr, _CPY)],
                                 ssem[u])
            pltpu.make_async_copy(rows[1], out_hbm.at[pl.ds(off, _CPY)],
                                  ssem[1]).wait()
            pltpu.make_async_copy(rows[0], out_hbm.at[pl.ds(off, _CPY)],
                                  ssem[0]).wait()

        @pl.when(t == _NTILES - 1)
        def _():
            tail0 = (_NTILES - 1) * _RPT
            tcpy = (_N - tail0) // 5
            for i in range(5):
                rr = tail0 + i * tcpy
                pltpu.sync_copy(agg.at[pl.ds(rr, tcpy)],
                                rows[0].at[pl.ds(0, tcpy)])
                pltpu.sync_copy(rows[0].at[pl.ds(0, tcpy)],
                                out_hbm.at[pl.ds(off + rr, tcpy)])


def _segsum(hflat, src, dst):
    kern = pl.kernel(
        _segsum_body,
        out_type=jax.ShapeDtypeStruct((_NCORES * _N, _D), jnp.float32),
        mesh=plsc.VectorSubcoreMesh(core_axis_name="c", subcore_axis_name="s"),
        scratch_types=[
            pltpu.VMEM((_CHUNK, _D), jnp.float32),
            pltpu.VMEM((_CHUNK, _D), jnp.float32),
            pltpu.VMEM((4, _CHUNK), jnp.int32),
            pltpu.VMEM((4, _CHUNK), jnp.int32),
            pltpu.SemaphoreType.DMA,
            pltpu.SemaphoreType.DMA,
            pltpu.SemaphoreType.DMA,
            pltpu.SemaphoreType.DMA,
            pltpu.SemaphoreType.DMA,
            pltpu.SemaphoreType.DMA,
            pltpu.SemaphoreType.DMA,
            pltpu.SemaphoreType.DMA,
            pltpu.VMEM_SHARED((_NPAD, _D), jnp.float32),
        ],
    )
    return kern(hflat, src, dst)


_BLK = 2000


def _mlp_body(h_ref, agg_ref, w1_ref, b1_ref, w2_ref, b2_ref, o_ref):
    z = jnp.dot(h_ref[0] + agg_ref[0], w1_ref[...],
                preferred_element_type=jnp.float32) + b1_ref[...]
    z = jnp.maximum(z, 0.0)
    o_ref[0] = jnp.dot(z, w2_ref[...],
                       preferred_element_type=jnp.float32) + b2_ref[...]


def _mlp(h, agg, w1, b1, w2, b2):
    nb = h.shape[0]
    return pl.pallas_call(
        _mlp_body,
        grid=(nb, _N // _BLK),
        in_specs=[
            pl.BlockSpec((1, _BLK, _D), lambda b, i: (b, i, 0)),
            pl.BlockSpec((1, _BLK, _D), lambda b, i: (b, i, 0)),
            pl.BlockSpec((_D, _D), lambda b, i: (0, 0)),
            pl.BlockSpec((1, _D), lambda b, i: (0, 0)),
            pl.BlockSpec((_D, _D), lambda b, i: (0, 0)),
            pl.BlockSpec((1, _D), lambda b, i: (0, 0)),
        ],
        out_specs=pl.BlockSpec((1, _BLK, _D), lambda b, i: (b, i, 0)),
        out_shape=jax.ShapeDtypeStruct((nb, _N, _D), jnp.float32),
    )(h, agg, w1, b1.reshape(1, _D), w2, b2.reshape(1, _D))


def _head_body(h_ref, agg_ref, w1_ref, b1_ref, w2_ref, b2_ref,
               cand_ref, act_ref, ms_ref, aw1_ref, ab1_ref,
               aw2_ref, ab2_ref, aw3_ref, ab3_ref, cw1_ref, cb1_ref,
               cw2_ref, cb2_ref, lp_ref, ent_ref, v_ref):
    b = pl.program_id(0)
    z = jnp.maximum(jnp.dot(h_ref[0] + agg_ref[0], w1_ref[...],
                            preferred_element_type=jnp.float32)
                    + b1_ref[...], 0.0)
    hb = jnp.dot(z, w2_ref[...],
                 preferred_element_type=jnp.float32) + b2_ref[...]
    g = jnp.mean(hb, axis=0, keepdims=True)
    base = (jnp.dot(g, aw1_ref[_D:2 * _D], preferred_element_type=jnp.float32)
            + jnp.dot(ms_ref[...], aw1_ref[2 * _D:3 * _D],
                      preferred_element_type=jnp.float32)
            + ab1_ref[...])
    s1 = jnp.maximum(jnp.dot(hb, aw1_ref[0:_D],
                             preferred_element_type=jnp.float32) + base, 0.0)
    s2 = jnp.maximum(jnp.dot(s1, aw2_ref[...],
                             preferred_element_type=jnp.float32)
                     + ab2_ref[...], 0.0)
    s = (jnp.dot(s2, aw3_ref[...], preferred_element_type=jnp.float32)
         + ab3_ref[0, 0]) * 10.0

    m0 = jnp.max(s)
    e0 = jnp.exp(s - m0)
    probs0 = e0 / jnp.sum(e0)

    neg_inf = jnp.float32(-jnp.inf)
    logits = jnp.where(cand_ref[0] > 0, probs0, neg_inf)
    m1 = jnp.max(logits)
    e1 = jnp.exp(logits - m1)
    z1 = jnp.sum(e1)
    logp = logits - (m1 + jnp.log(z1))

    a = act_ref[b]
    row = lax.broadcasted_iota(jnp.int32, (_N, 1), 0)
    lp = jnp.sum(jnp.where(row == a, logp, 0.0))
    p = e1 / z1
    fmin = jnp.finfo(jnp.float32).min
    ent = -jnp.sum(p * jnp.maximum(logp, fmin))

    hv = jnp.maximum(jnp.dot(g, cw1_ref[...],
                             preferred_element_type=jnp.float32)
                     + cb1_ref[...], 0.0)
    v = jnp.sum(jnp.dot(hv, cw2_ref[...],
                        preferred_element_type=jnp.float32)) + cb2_ref[0, 0]

    lp_ref[b] = lp
    ent_ref[b] = ent
    v_ref[b] = v


def _head(h, agg, w1, b1, w2, b2, candidates, action, machine_state,
          aw1, ab1, aw2, ab2, aw3, ab3, cw1, cb1, cw2, cb2):
    nb = h.shape[0]
    full = lambda b: (b, 0, 0)
    w0 = lambda b: (0, 0)
    return pl.pallas_call(
        _head_body,
        grid=(nb,),
        in_specs=[
            pl.BlockSpec((1, _N, _D), full),
            pl.BlockSpec((1, _N, _D), full),
            pl.BlockSpec((_D, _D), w0),
            pl.BlockSpec((1, _D), w0),
            pl.BlockSpec((_D, _D), w0),
            pl.BlockSpec((1, _D), w0),
            pl.BlockSpec((1, _N, 1), full),
            pl.BlockSpec(memory_space=pltpu.SMEM),
            pl.BlockSpec((1, _D), w0),
            pl.BlockSpec((3 * _D, _D), w0),
            pl.BlockSpec((1, _D), w0),
            pl.BlockSpec((_D, _D), w0),
            pl.BlockSpec((1, _D), w0),
            pl.BlockSpec((_D, 1), w0),
            pl.BlockSpec((1, 1), w0),
            pl.BlockSpec((_D, _CH), w0),
            pl.BlockSpec((1, _CH), w0),
            pl.BlockSpec((_CH, 1), w0),
            pl.BlockSpec((1, 1), w0),
        ],
        out_specs=[
            pl.BlockSpec(memory_space=pltpu.SMEM),
            pl.BlockSpec(memory_space=pltpu.SMEM),
            pl.BlockSpec(memory_space=pltpu.SMEM),
        ],
        out_shape=[
            jax.ShapeDtypeStruct((nb,), jnp.float32),
            jax.ShapeDtypeStruct((nb,), jnp.float32),
            jax.ShapeDtypeStruct((nb,), jnp.float32),
        ],
    )(h, agg, w1, b1.reshape(1, _D), w2, b2.reshape(1, _D),
      candidates.reshape(nb, _N, 1), action,
      machine_state.reshape(1, _D), aw1, ab1.reshape(1, _D),
      aw2, ab2.reshape(1, _D), aw3, ab3.reshape(1, 1),
      cw1, cb1.reshape(1, _CH), cw2, cb2.reshape(1, 1))


def kernel(x, edge_index, candidates, action, machine_state,
           g0w1, g0b1, g0w2, g0b2, g1w1, g1b1, g1w2, g1b2,
           g2w1, g2b1, g2w2, g2b2, aw1, ab1, aw2, ab2, aw3, ab3,
           cw1, cb1, cw2, cb2):
    src, dst = edge_index[0], edge_index[1]

    pairs = [x[0:2], x[2:4]]
    for (w1, b1, w2, b2) in ((g0w1, g0b1, g0w2, g0b2),
                             (g1w1, g1b1, g1w2, g1b2)):
        aggs = [_segsum(h.reshape(2 * _N, _D), src,
                        dst).reshape(2, _N, _D)
                for h in pairs]
        pairs = [_mlp(h, agg, w1, b1, w2, b2)
                 for h, agg in zip(pairs, aggs)]

    aggs = [_segsum(h.reshape(2 * _N, _D), src,
                    dst).reshape(2, _N, _D)
            for h in pairs]
    outs = []
    for p in range(2):
        outs.append(_head(pairs[p], aggs[p], g2w1, g2b1, g2w2, g2b2,
                          candidates[2 * p:2 * p + 2],
                          action[2 * p:2 * p + 2], machine_state,
                          aw1, ab1, aw2, ab2, aw3, ab3, cw1, cb1, cw2, cb2))
    lps = jnp.concatenate([outs[0][0], outs[1][0]])
    ents = jnp.concatenate([outs[0][1], outs[1][1]])
    vs = jnp.concatenate([outs[0][2], outs[1][2]])
    return action, lps, ents, vs
